# baseline (speedup 1.0000x reference)
import functools

import jax
import jax.numpy as jnp
from jax import lax
from jax.experimental import pallas as pl
from jax.experimental.pallas import tpu as pltpu
from jax.experimental.pallas import tpu_sc as plsc

B = 8
N = 1024
D = 32
CODES = 8192
CHUNK = 1024
NCHUNK = CODES // CHUNK

SC_CORES = 2
SC_SUBCORES = 16
NW = SC_CORES * SC_SUBCORES
ROWS_PER_W = (B * N) // NW
IDX_CHUNK = 128
DPAD = 128


def _assign_body(x_ref, e_ref, x2_ref, e2_ref, o_ref):
    x_t = x_ref[0]
    x2 = x2_ref[0]

    def step(j, carry):
        best_m, best_i = carry
        e = e_ref[j]
        e2 = e2_ref[j]
        xe = lax.dot_general(
            x_t, e, (((1,), (1,)), ((), ())),
            preferred_element_type=jnp.float32)
        d2 = jnp.maximum((x2 + e2) - 2.0 * xe, 0.0)
        dist = -jnp.sqrt(d2)
        m = jnp.max(dist, axis=1, keepdims=True)
        lanes = lax.broadcasted_iota(jnp.int32, dist.shape, 1)
        cand = jnp.min(jnp.where(dist == m, lanes, CHUNK),
                       axis=1, keepdims=True) + j * CHUNK
        upd = m > best_m
        return jnp.where(upd, m, best_m), jnp.where(upd, cand, best_i)

    init = (jnp.full((N, 1), -jnp.inf, jnp.float32),
            jnp.zeros((N, 1), jnp.int32))
    m0, i0 = lax.fori_loop(0, NCHUNK // 2, step, init)
    m1, i1 = lax.fori_loop(NCHUNK // 2, NCHUNK, step, init)
    m0r = m0.astype(jnp.bfloat16).astype(jnp.float32)
    keep0 = m0r >= m1
    o_ref[0] = jnp.where(keep0, i0, i1)


_assign = pl.pallas_call(
    _assign_body,
    grid=(B,),
    in_specs=[
        pl.BlockSpec((1, N, D), lambda i: (i, 0, 0)),
        pl.BlockSpec((NCHUNK, CHUNK, D), lambda i: (0, 0, 0)),
        pl.BlockSpec((1, N, 1), lambda i: (i, 0, 0)),
        pl.BlockSpec((NCHUNK, 1, CHUNK), lambda i: (0, 0, 0)),
    ],
    out_specs=pl.BlockSpec((1, N, 1), lambda i: (i, 0, 0)),
    out_shape=jax.ShapeDtypeStruct((B, N, 1), jnp.int32),
)


@functools.cache
def _make_gather():
    mesh = plsc.VectorSubcoreMesh(core_axis_name="c", subcore_axis_name="s")

    @functools.partial(
        pl.kernel, mesh=mesh,
        out_type=jax.ShapeDtypeStruct((B * N, DPAD), jnp.float32),
        scratch_types=[
            pltpu.VMEM((IDX_CHUNK,), jnp.int32),
            pltpu.VMEM((IDX_CHUNK, DPAD), jnp.float32),
            pltpu.SemaphoreType.DMA,
        ],
    )
    def gather(table_hbm, idx_hbm, out_hbm, idx_v, rows_v, sem):
        wid = lax.axis_index("s") * SC_CORES + lax.axis_index("c")
        base = wid * ROWS_PER_W
        for t in range(ROWS_PER_W // IDX_CHUNK):
            off = base + t * IDX_CHUNK
            pltpu.sync_copy(idx_hbm.at[pl.ds(off, IDX_CHUNK)], idx_v)
            pltpu.async_copy(table_hbm.at[idx_v], rows_v, sem).wait()
            pltpu.sync_copy(rows_v, out_hbm.at[pl.ds(off, IDX_CHUNK)])

    return gather


def kernel(x, embed):
    table = embed.reshape(CODES, D)
    flat = x.reshape(1, B * N, D)
    x2 = jnp.sum(flat * flat, axis=-1, keepdims=True).reshape(B, N, 1)
    e2 = jnp.sum(embed * embed, axis=-1).reshape(NCHUNK, 1, CHUNK)
    idx3 = _assign(x, table.reshape(NCHUNK, CHUNK, D), x2, e2)
    idx = idx3.reshape(B * N)
    table_pad = jnp.pad(table, ((0, 0), (0, DPAD - D)))
    quantize = _make_gather()(table_pad, idx)[:, :D].reshape(B, N, D)
    return quantize, idx3.reshape(B, N)


# speedup vs baseline: 1.2276x; 1.2276x over previous
import functools

import jax
import jax.numpy as jnp
from jax import lax
from jax.experimental import pallas as pl
from jax.experimental.pallas import tpu as pltpu
from jax.experimental.pallas import tpu_sc as plsc

B = 8
N = 1024
D = 32
CODES = 8192
CHUNK = 1024
NCHUNK = CODES // CHUNK

SC_CORES = 2
SC_SUBCORES = 16
NW = SC_CORES * SC_SUBCORES
ROWS_PER_W = (B * N) // NW
IDX_CHUNK = 128
DPAD = 128


def _assign_body(x_ref, e_ref, x2_ref, e2_ref, o_ref):
    x2r = x2_ref[0]
    x_t2 = x_ref[0] * 2.0

    def step(j, carry):
        best_s, best_i = carry
        e = e_ref[j]
        e2 = e2_ref[j]
        xe2 = lax.dot_general(
            e, x_t2, (((1,), (1,)), ((), ())),
            preferred_element_type=jnp.float32)
        d2 = jnp.maximum((x2r + e2) - xe2, 0.0)
        s = jnp.sqrt(d2)
        m = jnp.min(s, axis=0, keepdims=True)
        rows = lax.broadcasted_iota(jnp.int32, s.shape, 0)
        cand = jnp.min(jnp.where(s == m, rows, CHUNK),
                       axis=0, keepdims=True) + j * CHUNK
        upd = m < best_s
        return jnp.where(upd, m, best_s), jnp.where(upd, cand, best_i)

    init = (jnp.full((1, N), jnp.inf, jnp.float32),
            jnp.zeros((1, N), jnp.int32))
    s0, i0 = lax.fori_loop(0, NCHUNK // 2, step, init)
    s1, i1 = lax.fori_loop(NCHUNK // 2, NCHUNK, step, init)
    s0r = s0.astype(jnp.bfloat16).astype(jnp.float32)
    keep0 = s0r <= s1
    o_ref[0] = jnp.where(keep0, i0, i1)


_assign = pl.pallas_call(
    _assign_body,
    grid=(B,),
    in_specs=[
        pl.BlockSpec((1, N, D), lambda i: (i, 0, 0)),
        pl.BlockSpec((NCHUNK, CHUNK, D), lambda i: (0, 0, 0)),
        pl.BlockSpec((1, 1, N), lambda i: (i, 0, 0)),
        pl.BlockSpec((NCHUNK, CHUNK, 1), lambda i: (0, 0, 0)),
    ],
    out_specs=pl.BlockSpec((1, 1, N), lambda i: (i, 0, 0)),
    out_shape=jax.ShapeDtypeStruct((B, 1, N), jnp.int32),
)


@functools.cache
def _make_gather():
    mesh = plsc.VectorSubcoreMesh(core_axis_name="c", subcore_axis_name="s")

    @functools.partial(
        pl.kernel, mesh=mesh,
        out_type=jax.ShapeDtypeStruct((B * N, DPAD), jnp.float32),
        scratch_types=[
            pltpu.VMEM((IDX_CHUNK,), jnp.int32),
            pltpu.VMEM((IDX_CHUNK, DPAD), jnp.float32),
            pltpu.SemaphoreType.DMA,
        ],
    )
    def gather(table_hbm, idx_hbm, out_hbm, idx_v, rows_v, sem):
        wid = lax.axis_index("s") * SC_CORES + lax.axis_index("c")
        base = wid * ROWS_PER_W
        for t in range(ROWS_PER_W // IDX_CHUNK):
            off = base + t * IDX_CHUNK
            pltpu.sync_copy(idx_hbm.at[pl.ds(off, IDX_CHUNK)], idx_v)
            pltpu.async_copy(table_hbm.at[idx_v], rows_v, sem).wait()
            pltpu.sync_copy(rows_v, out_hbm.at[pl.ds(off, IDX_CHUNK)])

    return gather


def kernel(x, embed):
    table = embed.reshape(CODES, D)
    flat = x.reshape(1, B * N, D)
    x2 = jnp.sum(flat * flat, axis=-1, keepdims=True).reshape(B, 1, N)
    e2 = jnp.sum(embed * embed, axis=-1).reshape(NCHUNK, CHUNK, 1)
    idx3 = _assign(x, table.reshape(NCHUNK, CHUNK, D), x2, e2)
    idx = idx3.reshape(B * N)
    table_pad = jnp.pad(table, ((0, 0), (0, DPAD - D)))
    quantize = _make_gather()(table_pad, idx)[:, :D].reshape(B, N, D)
    return quantize, idx3.reshape(B, N)


# speedup vs baseline: 1.2919x; 1.0525x over previous
import functools

import jax
import jax.numpy as jnp
from jax import lax
from jax.experimental import pallas as pl
from jax.experimental.pallas import tpu as pltpu
from jax.experimental.pallas import tpu_sc as plsc

B = 8
N = 1024
D = 32
CODES = 8192
CHUNK = 1024
NCHUNK = CODES // CHUNK

SC_CORES = 2
SC_SUBCORES = 16
NW = SC_CORES * SC_SUBCORES
ROWS_PER_W = (B * N) // NW
IDX_CHUNK = 128
DPAD = 128


HALF = NCHUNK // 2
TIE_REL = float(jnp.float32(1.0 + 2.0 ** -17))


def _assign_body(x_ref, e_ref, x2_ref, e2_ref, o_ref, d2_scr):
    x2r = x2_ref[0]
    x_t2 = x_ref[0] * 2.0

    def half(h):
        def pass1(j, acc):
            e = e_ref[h * HALF + j]
            e2 = e2_ref[h * HALF + j]
            xe2 = lax.dot_general(
                e, x_t2, (((1,), (1,)), ((), ())),
                preferred_element_type=jnp.float32)
            d2 = (x2r + e2) - xe2
            d2_scr[pl.ds(j * CHUNK, CHUNK), :] = d2
            return jnp.minimum(acc, jnp.min(d2, axis=0, keepdims=True))

        d2min = lax.fori_loop(0, HALF, pass1,
                              jnp.full((1, N), jnp.inf, jnp.float32))
        s_fast = jnp.sqrt(jnp.maximum(d2min, 0.0))
        thr = jnp.where(d2min > 0.0, d2min * TIE_REL, 1.2e-38)

        def pass2(j, carry):
            acc_i, acc_c = carry
            d2 = d2_scr[pl.ds(j * CHUNK, CHUNK), :]
            mask = d2 <= thr
            rows = lax.broadcasted_iota(jnp.int32, d2.shape, 0)
            cand = jnp.min(jnp.where(mask, rows + (h * HALF + j) * CHUNK,
                                     CODES), axis=0, keepdims=True)
            cnt = jnp.sum(jnp.where(mask, 1.0, 0.0), axis=0, keepdims=True)
            return jnp.minimum(acc_i, cand), acc_c + cnt

        fi, cnt = lax.fori_loop(
            0, HALF, pass2,
            (jnp.full((1, N), CODES, jnp.int32),
             jnp.zeros((1, N), jnp.float32)))
        need_slow = jnp.max(cnt) > 1.5

        def slow(_):
            def sstep(j, carry):
                best_s, best_i = carry
                d2 = d2_scr[pl.ds(j * CHUNK, CHUNK), :]
                s_el = jnp.sqrt(jnp.maximum(d2, 0.0))
                m = jnp.min(s_el, axis=0, keepdims=True)
                rows = lax.broadcasted_iota(jnp.int32, s_el.shape, 0)
                cand = jnp.min(jnp.where(s_el == m,
                                         rows + (h * HALF + j) * CHUNK,
                                         CODES), axis=0, keepdims=True)
                upd = m < best_s
                return (jnp.where(upd, m, best_s),
                        jnp.where(upd, cand, best_i))

            return lax.fori_loop(0, HALF, sstep,
                                 (jnp.full((1, N), jnp.inf, jnp.float32),
                                  jnp.full((1, N), CODES, jnp.int32)))

        return lax.cond(need_slow, slow, lambda _: (s_fast, fi), None)

    s0, i0 = half(0)
    s1, i1 = half(1)
    s0r = s0.astype(jnp.bfloat16).astype(jnp.float32)
    keep0 = s0r <= s1
    o_ref[0] = jnp.where(keep0, i0, i1)


_assign = pl.pallas_call(
    _assign_body,
    grid=(B,),
    in_specs=[
        pl.BlockSpec((1, N, D), lambda i: (i, 0, 0)),
        pl.BlockSpec((NCHUNK, CHUNK, D), lambda i: (0, 0, 0)),
        pl.BlockSpec((1, 1, N), lambda i: (i, 0, 0)),
        pl.BlockSpec((NCHUNK, CHUNK, 1), lambda i: (0, 0, 0)),
    ],
    out_specs=pl.BlockSpec((1, 1, N), lambda i: (i, 0, 0)),
    out_shape=jax.ShapeDtypeStruct((B, 1, N), jnp.int32),
    scratch_shapes=[pltpu.VMEM((HALF * CHUNK, N), jnp.float32)],
)


@functools.cache
def _make_gather():
    mesh = plsc.VectorSubcoreMesh(core_axis_name="c", subcore_axis_name="s")

    @functools.partial(
        pl.kernel, mesh=mesh,
        out_type=jax.ShapeDtypeStruct((B * N, DPAD), jnp.float32),
        scratch_types=[
            pltpu.VMEM((IDX_CHUNK,), jnp.int32),
            pltpu.VMEM((IDX_CHUNK, DPAD), jnp.float32),
            pltpu.SemaphoreType.DMA,
        ],
    )
    def gather(table_hbm, idx_hbm, out_hbm, idx_v, rows_v, sem):
        wid = lax.axis_index("s") * SC_CORES + lax.axis_index("c")
        base = wid * ROWS_PER_W
        for t in range(ROWS_PER_W // IDX_CHUNK):
            off = base + t * IDX_CHUNK
            pltpu.sync_copy(idx_hbm.at[pl.ds(off, IDX_CHUNK)], idx_v)
            pltpu.async_copy(table_hbm.at[idx_v], rows_v, sem).wait()
            pltpu.sync_copy(rows_v, out_hbm.at[pl.ds(off, IDX_CHUNK)])

    return gather


def kernel(x, embed):
    table = embed.reshape(CODES, D)
    flat = x.reshape(1, B * N, D)
    x2 = jnp.sum(flat * flat, axis=-1, keepdims=True).reshape(B, 1, N)
    e2 = jnp.sum(embed * embed, axis=-1).reshape(NCHUNK, CHUNK, 1)
    idx3 = _assign(x, table.reshape(NCHUNK, CHUNK, D), x2, e2)
    idx = idx3.reshape(B * N)
    table_pad = jnp.pad(table, ((0, 0), (0, DPAD - D)))
    quantize = _make_gather()(table_pad, idx)[:, :D].reshape(B, N, D)
    return quantize, idx3.reshape(B, N)


# speedup vs baseline: 1.3432x; 1.0397x over previous
import functools

import jax
import jax.numpy as jnp
from jax import lax
from jax.experimental import pallas as pl
from jax.experimental.pallas import tpu as pltpu
from jax.experimental.pallas import tpu_sc as plsc

B = 8
N = 1024
D = 32
CODES = 8192
CHUNK = 1024
NCHUNK = CODES // CHUNK

SC_CORES = 2
SC_SUBCORES = 16
NW = SC_CORES * SC_SUBCORES
ROWS_PER_W = (B * N) // NW
IDX_CHUNK = 128
DPAD = 128


HALF = NCHUNK // 2
TIE_REL = 1.0 + 2.0 ** -17


def _assign_body(x_ref, e_ref, x2_ref, e2_ref, o_ref, d2_scr):
    x2r = x2_ref[0]
    x_t2 = x_ref[0] * 2.0

    def half(h):
        def pass1(j, carry):
            m_acc, fi_acc, cnt_acc = carry
            e = e_ref[h * HALF + j]
            e2 = e2_ref[h * HALF + j]
            xe2 = lax.dot_general(
                e, x_t2, (((1,), (1,)), ((), ())),
                preferred_element_type=jnp.float32)
            d2 = (x2r + e2) - xe2
            d2_scr[pl.ds(j * CHUNK, CHUNK), :] = d2
            m_c = jnp.min(d2, axis=0, keepdims=True)
            thr_c = jnp.where(m_c > 0.0, m_c * TIE_REL, 1.2e-38)
            mask = d2 <= thr_c
            rows = lax.broadcasted_iota(jnp.int32, d2.shape, 0)
            fi_c = jnp.min(jnp.where(mask, rows + (h * HALF + j) * CHUNK,
                                     CODES), axis=0, keepdims=True)
            cnt_c = jnp.sum(jnp.where(mask, 1.0, 0.0), axis=0, keepdims=True)
            m_new = jnp.minimum(m_acc, m_c)
            thr = jnp.where(m_new > 0.0, m_new * TIE_REL, 1.2e-38)
            in_a = m_acc <= thr
            in_c = m_c <= thr
            cnt_new = (jnp.where(in_a, cnt_acc, 0.0)
                       + jnp.where(in_c, cnt_c, 0.0))
            fi_new = jnp.minimum(jnp.where(in_a, fi_acc, CODES),
                                 jnp.where(in_c, fi_c, CODES))
            return m_new, fi_new, cnt_new

        d2min, fi, cnt = lax.fori_loop(
            0, HALF, pass1,
            (jnp.full((1, N), jnp.inf, jnp.float32),
             jnp.full((1, N), CODES, jnp.int32),
             jnp.zeros((1, N), jnp.float32)))
        s_fast = jnp.sqrt(jnp.maximum(d2min, 0.0))
        need_slow = jnp.max(cnt) > 1.5

        def slow(_):
            def sstep(j, carry):
                best_s, best_i = carry
                d2 = d2_scr[pl.ds(j * CHUNK, CHUNK), :]
                s_el = jnp.sqrt(jnp.maximum(d2, 0.0))
                m = jnp.min(s_el, axis=0, keepdims=True)
                rows = lax.broadcasted_iota(jnp.int32, s_el.shape, 0)
                cand = jnp.min(jnp.where(s_el == m,
                                         rows + (h * HALF + j) * CHUNK,
                                         CODES), axis=0, keepdims=True)
                upd = m < best_s
                return (jnp.where(upd, m, best_s),
                        jnp.where(upd, cand, best_i))

            return lax.fori_loop(0, HALF, sstep,
                                 (jnp.full((1, N), jnp.inf, jnp.float32),
                                  jnp.full((1, N), CODES, jnp.int32)))

        return lax.cond(need_slow, slow, lambda _: (s_fast, fi), None)

    s0, i0 = half(0)
    s1, i1 = half(1)
    s0r = s0.astype(jnp.bfloat16).astype(jnp.float32)
    keep0 = s0r <= s1
    o_ref[0] = jnp.where(keep0, i0, i1)


_assign = pl.pallas_call(
    _assign_body,
    grid=(B,),
    in_specs=[
        pl.BlockSpec((1, N, D), lambda i: (i, 0, 0)),
        pl.BlockSpec((NCHUNK, CHUNK, D), lambda i: (0, 0, 0)),
        pl.BlockSpec((1, 1, N), lambda i: (i, 0, 0)),
        pl.BlockSpec((NCHUNK, CHUNK, 1), lambda i: (0, 0, 0)),
    ],
    out_specs=pl.BlockSpec((1, 1, N), lambda i: (i, 0, 0)),
    out_shape=jax.ShapeDtypeStruct((B, 1, N), jnp.int32),
    scratch_shapes=[pltpu.VMEM((HALF * CHUNK, N), jnp.float32)],
)


@functools.cache
def _make_gather():
    mesh = plsc.VectorSubcoreMesh(core_axis_name="c", subcore_axis_name="s")

    @functools.partial(
        pl.kernel, mesh=mesh,
        out_type=jax.ShapeDtypeStruct((B * N, DPAD), jnp.float32),
        scratch_types=[
            pltpu.VMEM((IDX_CHUNK,), jnp.int32),
            pltpu.VMEM((IDX_CHUNK, DPAD), jnp.float32),
            pltpu.SemaphoreType.DMA,
        ],
    )
    def gather(table_hbm, idx_hbm, out_hbm, idx_v, rows_v, sem):
        wid = lax.axis_index("s") * SC_CORES + lax.axis_index("c")
        base = wid * ROWS_PER_W
        for t in range(ROWS_PER_W // IDX_CHUNK):
            off = base + t * IDX_CHUNK
            pltpu.sync_copy(idx_hbm.at[pl.ds(off, IDX_CHUNK)], idx_v)
            pltpu.async_copy(table_hbm.at[idx_v], rows_v, sem).wait()
            pltpu.sync_copy(rows_v, out_hbm.at[pl.ds(off, IDX_CHUNK)])

    return gather


def kernel(x, embed):
    table = embed.reshape(CODES, D)
    flat = x.reshape(1, B * N, D)
    x2 = jnp.sum(flat * flat, axis=-1, keepdims=True).reshape(B, 1, N)
    e2 = jnp.sum(embed * embed, axis=-1).reshape(NCHUNK, CHUNK, 1)
    idx3 = _assign(x, table.reshape(NCHUNK, CHUNK, D), x2, e2)
    idx = idx3.reshape(B * N)
    table_pad = jnp.pad(table, ((0, 0), (0, DPAD - D)))
    quantize = _make_gather()(table_pad, idx)[:, :D].reshape(B, N, D)
    return quantize, idx3.reshape(B, N)


# speedup vs baseline: 1.4007x; 1.0428x over previous
import functools

import jax
import jax.numpy as jnp
from jax import lax
from jax.experimental import pallas as pl
from jax.experimental.pallas import tpu as pltpu
from jax.experimental.pallas import tpu_sc as plsc

B = 8
N = 1024
D = 32
CODES = 8192
CHUNK = 1024
NCHUNK = CODES // CHUNK

SC_CORES = 2
SC_SUBCORES = 16
NW = SC_CORES * SC_SUBCORES
ROWS_PER_W = (B * N) // NW
IDX_CHUNK = 128
DPAD = 128


HALF = NCHUNK // 2
TIE_REL = 1.0 + 2.0 ** -17


def _assign_body(x_ref, e_ref, x2_ref, e2_ref, o_ref, d2_scr):
    x2r = x2_ref[0]
    x_t2 = x_ref[0] * 2.0
    rows = lax.broadcasted_iota(jnp.int32, (CHUNK, N), 0)

    def half(h):
        def pass1(j, carry):
            m_acc, fi_acc, cnt_acc = carry
            e = e_ref[h * HALF + j]
            e2 = e2_ref[h * HALF + j]
            xe2 = lax.dot_general(
                e, x_t2, (((1,), (1,)), ((), ())),
                preferred_element_type=jnp.float32)
            d2 = (x2r + e2) - xe2
            d2_scr[pl.ds(j * CHUNK, CHUNK), :] = d2
            m_c = jnp.min(d2, axis=0, keepdims=True)
            thr_c = jnp.where(m_c > 0.0, m_c * TIE_REL, 1.2e-38)
            mask = d2 <= thr_c
            fi_c = (jnp.min(jnp.where(mask, rows, CHUNK),
                            axis=0, keepdims=True) + (h * HALF + j) * CHUNK)
            cnt_c = jnp.sum(jnp.where(mask, 1.0, 0.0), axis=0, keepdims=True)
            m_new = jnp.minimum(m_acc, m_c)
            thr = jnp.where(m_new > 0.0, m_new * TIE_REL, 1.2e-38)
            in_a = m_acc <= thr
            in_c = m_c <= thr
            cnt_new = (jnp.where(in_a, cnt_acc, 0.0)
                       + jnp.where(in_c, cnt_c, 0.0))
            fi_new = jnp.minimum(jnp.where(in_a, fi_acc, CODES),
                                 jnp.where(in_c, fi_c, CODES))
            return m_new, fi_new, cnt_new

        d2min, fi, cnt = lax.fori_loop(
            0, HALF, pass1,
            (jnp.full((1, N), jnp.inf, jnp.float32),
             jnp.full((1, N), CODES, jnp.int32),
             jnp.zeros((1, N), jnp.float32)))
        s_fast = jnp.sqrt(jnp.maximum(d2min, 0.0))
        need_slow = jnp.max(cnt) > 1.5

        def slow(_):
            def sstep(j, carry):
                best_s, best_i = carry
                d2 = d2_scr[pl.ds(j * CHUNK, CHUNK), :]
                s_el = jnp.sqrt(jnp.maximum(d2, 0.0))
                m = jnp.min(s_el, axis=0, keepdims=True)
                rows = lax.broadcasted_iota(jnp.int32, s_el.shape, 0)
                cand = jnp.min(jnp.where(s_el == m,
                                         rows + (h * HALF + j) * CHUNK,
                                         CODES), axis=0, keepdims=True)
                upd = m < best_s
                return (jnp.where(upd, m, best_s),
                        jnp.where(upd, cand, best_i))

            return lax.fori_loop(0, HALF, sstep,
                                 (jnp.full((1, N), jnp.inf, jnp.float32),
                                  jnp.full((1, N), CODES, jnp.int32)))

        return lax.cond(need_slow, slow, lambda _: (s_fast, fi), None)

    s0, i0 = half(0)
    s1, i1 = half(1)
    s0r = s0.astype(jnp.bfloat16).astype(jnp.float32)
    keep0 = s0r <= s1
    o_ref[0] = jnp.where(keep0, i0, i1)


_assign = pl.pallas_call(
    _assign_body,
    grid=(B,),
    in_specs=[
        pl.BlockSpec((1, N, D), lambda i: (i, 0, 0)),
        pl.BlockSpec((NCHUNK, CHUNK, D), lambda i: (0, 0, 0)),
        pl.BlockSpec((1, 1, N), lambda i: (i, 0, 0)),
        pl.BlockSpec((NCHUNK, CHUNK, 1), lambda i: (0, 0, 0)),
    ],
    out_specs=pl.BlockSpec((1, 1, N), lambda i: (i, 0, 0)),
    out_shape=jax.ShapeDtypeStruct((B, 1, N), jnp.int32),
    scratch_shapes=[pltpu.VMEM((HALF * CHUNK, N), jnp.float32)],
)


@functools.cache
def _make_gather():
    mesh = plsc.VectorSubcoreMesh(core_axis_name="c", subcore_axis_name="s")

    @functools.partial(
        pl.kernel, mesh=mesh,
        out_type=jax.ShapeDtypeStruct((B * N, DPAD), jnp.float32),
        scratch_types=[
            pltpu.VMEM((IDX_CHUNK,), jnp.int32),
            pltpu.VMEM((IDX_CHUNK, DPAD), jnp.float32),
            pltpu.SemaphoreType.DMA,
        ],
    )
    def gather(table_hbm, idx_hbm, out_hbm, idx_v, rows_v, sem):
        wid = lax.axis_index("s") * SC_CORES + lax.axis_index("c")
        base = wid * ROWS_PER_W
        for t in range(ROWS_PER_W // IDX_CHUNK):
            off = base + t * IDX_CHUNK
            pltpu.sync_copy(idx_hbm.at[pl.ds(off, IDX_CHUNK)], idx_v)
            pltpu.async_copy(table_hbm.at[idx_v], rows_v, sem).wait()
            pltpu.sync_copy(rows_v, out_hbm.at[pl.ds(off, IDX_CHUNK)])

    return gather


def kernel(x, embed):
    table = embed.reshape(CODES, D)
    flat = x.reshape(1, B * N, D)
    x2 = jnp.sum(flat * flat, axis=-1, keepdims=True).reshape(B, 1, N)
    e2 = jnp.sum(embed * embed, axis=-1).reshape(NCHUNK, CHUNK, 1)
    idx3 = _assign(x, table.reshape(NCHUNK, CHUNK, D), x2, e2)
    idx = idx3.reshape(B * N)
    table_pad = jnp.pad(table, ((0, 0), (0, DPAD - D)))
    quantize = _make_gather()(table_pad, idx)[:, :D].reshape(B, N, D)
    return quantize, idx3.reshape(B, N)


# speedup vs baseline: 1.4243x; 1.0169x over previous
import functools

import jax
import jax.numpy as jnp
from jax import lax
from jax.experimental import pallas as pl
from jax.experimental.pallas import tpu as pltpu
from jax.experimental.pallas import tpu_sc as plsc

B = 8
N = 1024
D = 32
CODES = 8192
CHUNK = 2048
NCHUNK = CODES // CHUNK

SC_CORES = 2
SC_SUBCORES = 16
NW = SC_CORES * SC_SUBCORES
ROWS_PER_W = (B * N) // NW
IDX_CHUNK = 128
DPAD = 128


HALF = NCHUNK // 2
TIE_REL = 1.0 + 2.0 ** -17


def _assign_body(x_ref, e_ref, x2_ref, e2_ref, o_ref, d2_scr):
    x2r = x2_ref[0]
    x_t2 = x_ref[0] * 2.0
    rows = lax.broadcasted_iota(jnp.int32, (CHUNK, N), 0)

    def half(h):
        def pass1(j, carry):
            m_acc, fi_acc, cnt_acc = carry
            e = e_ref[h * HALF + j]
            e2 = e2_ref[h * HALF + j]
            xe2 = lax.dot_general(
                e, x_t2, (((1,), (1,)), ((), ())),
                preferred_element_type=jnp.float32)
            d2 = (x2r + e2) - xe2
            d2_scr[pl.ds(j * CHUNK, CHUNK), :] = d2
            m_c = jnp.min(d2, axis=0, keepdims=True)
            thr_c = jnp.where(m_c > 0.0, m_c * TIE_REL, 1.2e-38)
            mask = d2 <= thr_c
            fi_c = (jnp.min(jnp.where(mask, rows, CHUNK),
                            axis=0, keepdims=True) + (h * HALF + j) * CHUNK)
            cnt_c = jnp.sum(jnp.where(mask, 1.0, 0.0), axis=0, keepdims=True)
            m_new = jnp.minimum(m_acc, m_c)
            thr = jnp.where(m_new > 0.0, m_new * TIE_REL, 1.2e-38)
            in_a = m_acc <= thr
            in_c = m_c <= thr
            cnt_new = (jnp.where(in_a, cnt_acc, 0.0)
                       + jnp.where(in_c, cnt_c, 0.0))
            fi_new = jnp.minimum(jnp.where(in_a, fi_acc, CODES),
                                 jnp.where(in_c, fi_c, CODES))
            return m_new, fi_new, cnt_new

        d2min, fi, cnt = lax.fori_loop(
            0, HALF, pass1,
            (jnp.full((1, N), jnp.inf, jnp.float32),
             jnp.full((1, N), CODES, jnp.int32),
             jnp.zeros((1, N), jnp.float32)))
        s_fast = jnp.sqrt(jnp.maximum(d2min, 0.0))
        need_slow = jnp.max(cnt) > 1.5

        def slow(_):
            def sstep(j, carry):
                best_s, best_i = carry
                d2 = d2_scr[pl.ds(j * CHUNK, CHUNK), :]
                s_el = jnp.sqrt(jnp.maximum(d2, 0.0))
                m = jnp.min(s_el, axis=0, keepdims=True)
                rows = lax.broadcasted_iota(jnp.int32, s_el.shape, 0)
                cand = jnp.min(jnp.where(s_el == m,
                                         rows + (h * HALF + j) * CHUNK,
                                         CODES), axis=0, keepdims=True)
                upd = m < best_s
                return (jnp.where(upd, m, best_s),
                        jnp.where(upd, cand, best_i))

            return lax.fori_loop(0, HALF, sstep,
                                 (jnp.full((1, N), jnp.inf, jnp.float32),
                                  jnp.full((1, N), CODES, jnp.int32)))

        return lax.cond(need_slow, slow, lambda _: (s_fast, fi), None)

    s0, i0 = half(0)
    s1, i1 = half(1)
    s0r = s0.astype(jnp.bfloat16).astype(jnp.float32)
    keep0 = s0r <= s1
    o_ref[0] = jnp.where(keep0, i0, i1)


_assign = pl.pallas_call(
    _assign_body,
    grid=(B,),
    in_specs=[
        pl.BlockSpec((1, N, D), lambda i: (i, 0, 0)),
        pl.BlockSpec((NCHUNK, CHUNK, D), lambda i: (0, 0, 0)),
        pl.BlockSpec((1, 1, N), lambda i: (i, 0, 0)),
        pl.BlockSpec((NCHUNK, CHUNK, 1), lambda i: (0, 0, 0)),
    ],
    out_specs=pl.BlockSpec((1, 1, N), lambda i: (i, 0, 0)),
    out_shape=jax.ShapeDtypeStruct((B, 1, N), jnp.int32),
    scratch_shapes=[pltpu.VMEM((HALF * CHUNK, N), jnp.float32)],
)


@functools.cache
def _make_gather():
    mesh = plsc.VectorSubcoreMesh(core_axis_name="c", subcore_axis_name="s")

    @functools.partial(
        pl.kernel, mesh=mesh,
        out_type=jax.ShapeDtypeStruct((B * N, DPAD), jnp.float32),
        scratch_types=[
            pltpu.VMEM((IDX_CHUNK,), jnp.int32),
            pltpu.VMEM((IDX_CHUNK, DPAD), jnp.float32),
            pltpu.SemaphoreType.DMA,
        ],
    )
    def gather(table_hbm, idx_hbm, out_hbm, idx_v, rows_v, sem):
        wid = lax.axis_index("s") * SC_CORES + lax.axis_index("c")
        base = wid * ROWS_PER_W
        for t in range(ROWS_PER_W // IDX_CHUNK):
            off = base + t * IDX_CHUNK
            pltpu.sync_copy(idx_hbm.at[pl.ds(off, IDX_CHUNK)], idx_v)
            pltpu.async_copy(table_hbm.at[idx_v], rows_v, sem).wait()
            pltpu.sync_copy(rows_v, out_hbm.at[pl.ds(off, IDX_CHUNK)])

    return gather


def kernel(x, embed):
    table = embed.reshape(CODES, D)
    flat = x.reshape(1, B * N, D)
    x2 = jnp.sum(flat * flat, axis=-1, keepdims=True).reshape(B, 1, N)
    e2 = jnp.sum(embed * embed, axis=-1).reshape(NCHUNK, CHUNK, 1)
    idx3 = _assign(x, table.reshape(NCHUNK, CHUNK, D), x2, e2)
    idx = idx3.reshape(B * N)
    table_pad = jnp.pad(table, ((0, 0), (0, DPAD - D)))
    quantize = _make_gather()(table_pad, idx)[:, :D].reshape(B, N, D)
    return quantize, idx3.reshape(B, N)


# speedup vs baseline: 1.4342x; 1.0069x over previous
import functools

import jax
import jax.numpy as jnp
from jax import lax
from jax.experimental import pallas as pl
from jax.experimental.pallas import tpu as pltpu
from jax.experimental.pallas import tpu_sc as plsc

B = 8
N = 1024
D = 32
CODES = 8192
CHUNK = 4096
NCHUNK = CODES // CHUNK

SC_CORES = 2
SC_SUBCORES = 16
NW = SC_CORES * SC_SUBCORES
ROWS_PER_W = (B * N) // NW
IDX_CHUNK = 128
DPAD = 128


HALF = NCHUNK // 2
TIE_REL = 1.0 + 2.0 ** -17


def _assign_body(x_ref, e_ref, x2_ref, e2_ref, o_ref, d2_scr):
    x2r = x2_ref[0]
    x_t2 = x_ref[0] * 2.0
    rows = lax.broadcasted_iota(jnp.int32, (CHUNK, N), 0)

    def half(h):
        def pass1(j, carry):
            m_acc, fi_acc, cnt_acc = carry
            e = e_ref[h * HALF + j]
            e2 = e2_ref[h * HALF + j]
            xe2 = lax.dot_general(
                e, x_t2, (((1,), (1,)), ((), ())),
                preferred_element_type=jnp.float32)
            d2 = (x2r + e2) - xe2
            d2_scr[pl.ds(j * CHUNK, CHUNK), :] = d2
            m_c = jnp.min(d2, axis=0, keepdims=True)
            thr_c = jnp.where(m_c > 0.0, m_c * TIE_REL, 1.2e-38)
            mask = d2 <= thr_c
            fi_c = (jnp.min(jnp.where(mask, rows, CHUNK),
                            axis=0, keepdims=True) + (h * HALF + j) * CHUNK)
            cnt_c = jnp.sum(jnp.where(mask, 1.0, 0.0), axis=0, keepdims=True)
            m_new = jnp.minimum(m_acc, m_c)
            thr = jnp.where(m_new > 0.0, m_new * TIE_REL, 1.2e-38)
            in_a = m_acc <= thr
            in_c = m_c <= thr
            cnt_new = (jnp.where(in_a, cnt_acc, 0.0)
                       + jnp.where(in_c, cnt_c, 0.0))
            fi_new = jnp.minimum(jnp.where(in_a, fi_acc, CODES),
                                 jnp.where(in_c, fi_c, CODES))
            return m_new, fi_new, cnt_new

        d2min, fi, cnt = lax.fori_loop(
            0, HALF, pass1,
            (jnp.full((1, N), jnp.inf, jnp.float32),
             jnp.full((1, N), CODES, jnp.int32),
             jnp.zeros((1, N), jnp.float32)))
        s_fast = jnp.sqrt(jnp.maximum(d2min, 0.0))
        need_slow = jnp.max(cnt) > 1.5

        def slow(_):
            def sstep(j, carry):
                best_s, best_i = carry
                d2 = d2_scr[pl.ds(j * CHUNK, CHUNK), :]
                s_el = jnp.sqrt(jnp.maximum(d2, 0.0))
                m = jnp.min(s_el, axis=0, keepdims=True)
                rows = lax.broadcasted_iota(jnp.int32, s_el.shape, 0)
                cand = jnp.min(jnp.where(s_el == m,
                                         rows + (h * HALF + j) * CHUNK,
                                         CODES), axis=0, keepdims=True)
                upd = m < best_s
                return (jnp.where(upd, m, best_s),
                        jnp.where(upd, cand, best_i))

            return lax.fori_loop(0, HALF, sstep,
                                 (jnp.full((1, N), jnp.inf, jnp.float32),
                                  jnp.full((1, N), CODES, jnp.int32)))

        return lax.cond(need_slow, slow, lambda _: (s_fast, fi), None)

    s0, i0 = half(0)
    s1, i1 = half(1)
    s0r = s0.astype(jnp.bfloat16).astype(jnp.float32)
    keep0 = s0r <= s1
    o_ref[0] = jnp.where(keep0, i0, i1)


_assign = pl.pallas_call(
    _assign_body,
    grid=(B,),
    in_specs=[
        pl.BlockSpec((1, N, D), lambda i: (i, 0, 0)),
        pl.BlockSpec((NCHUNK, CHUNK, D), lambda i: (0, 0, 0)),
        pl.BlockSpec((1, 1, N), lambda i: (i, 0, 0)),
        pl.BlockSpec((NCHUNK, CHUNK, 1), lambda i: (0, 0, 0)),
    ],
    out_specs=pl.BlockSpec((1, 1, N), lambda i: (i, 0, 0)),
    out_shape=jax.ShapeDtypeStruct((B, 1, N), jnp.int32),
    scratch_shapes=[pltpu.VMEM((HALF * CHUNK, N), jnp.float32)],
)


@functools.cache
def _make_gather():
    mesh = plsc.VectorSubcoreMesh(core_axis_name="c", subcore_axis_name="s")

    @functools.partial(
        pl.kernel, mesh=mesh,
        out_type=jax.ShapeDtypeStruct((B * N, DPAD), jnp.float32),
        scratch_types=[
            pltpu.VMEM((IDX_CHUNK,), jnp.int32),
            pltpu.VMEM((IDX_CHUNK, DPAD), jnp.float32),
            pltpu.SemaphoreType.DMA,
        ],
    )
    def gather(table_hbm, idx_hbm, out_hbm, idx_v, rows_v, sem):
        wid = lax.axis_index("s") * SC_CORES + lax.axis_index("c")
        base = wid * ROWS_PER_W
        for t in range(ROWS_PER_W // IDX_CHUNK):
            off = base + t * IDX_CHUNK
            pltpu.sync_copy(idx_hbm.at[pl.ds(off, IDX_CHUNK)], idx_v)
            pltpu.async_copy(table_hbm.at[idx_v], rows_v, sem).wait()
            pltpu.sync_copy(rows_v, out_hbm.at[pl.ds(off, IDX_CHUNK)])

    return gather


def kernel(x, embed):
    table = embed.reshape(CODES, D)
    flat = x.reshape(1, B * N, D)
    x2 = jnp.sum(flat * flat, axis=-1, keepdims=True).reshape(B, 1, N)
    e2 = jnp.sum(embed * embed, axis=-1).reshape(NCHUNK, CHUNK, 1)
    idx3 = _assign(x, table.reshape(NCHUNK, CHUNK, D), x2, e2)
    idx = idx3.reshape(B * N)
    table_pad = jnp.pad(table, ((0, 0), (0, DPAD - D)))
    quantize = _make_gather()(table_pad, idx)[:, :D].reshape(B, N, D)
    return quantize, idx3.reshape(B, N)
